# baseline (speedup 1.0000x reference)
import functools
import jax
import jax.numpy as jnp
from jax import lax
from jax.experimental import pallas as pl
from jax.experimental.pallas import tpu as pltpu

_TV = 4096

def _project_tc(emb, W, b2d):
    B, H = emb.shape
    V = W.shape[0]
    nv = pl.cdiv(V, _TV)

    def mm_kernel(emb_ref, w_ref, b_ref, out_ref):
        wt = w_ref[...].T
        out_ref[...] = (
            jnp.dot(emb_ref[...], wt, preferred_element_type=jnp.float32)
            + b_ref[...]
        )

    return pl.pallas_call(
        mm_kernel,
        grid=(nv,),
        in_specs=[
            pl.BlockSpec((B, H), lambda i: (0, 0)),
            pl.BlockSpec((_TV, H), lambda i: (i, 0)),
            pl.BlockSpec((1, _TV), lambda i: (0, i)),
        ],
        out_specs=pl.BlockSpec((B, _TV), lambda i: (0, i)),
        out_shape=jax.ShapeDtypeStruct((B, V), jnp.float32),
    )(emb, W, b2d)


def kernel(x, table, W, b):
    V, H = W.shape
    emb = jnp.take(table, x, axis=0)
    return _project_tc(emb, W, b.reshape(1, V))
